# baseline (speedup 1.0000x reference)
import dataclasses
import functools

import jax
import jax.numpy as jnp
from jax import lax
from jax.experimental import pallas as pl
from jax.experimental.pallas import tpu as pltpu
from jax.experimental.pallas import tpu_sc as plsc

_NC = 2
_NS = 16
_NW = _NC * _NS


def _sc_params():
    cp = pltpu.CompilerParams()
    if "needs_layout_passes" in pltpu.CompilerParams.__dataclass_fields__:
        cp = dataclasses.replace(cp, needs_layout_passes=False)
    return cp


def _edge_head_sc(P, Q, row, col, w2, b2v):
    E_ = row.shape[0]
    H_ = P.shape[1]
    C = 200
    per_w = E_ // _NW
    n_chunks = per_w // C
    mesh = plsc.VectorSubcoreMesh(core_axis_name="c", subcore_axis_name="s")

    @functools.partial(
        pl.kernel, mesh=mesh,
        out_type=jax.ShapeDtypeStruct((E_,), jnp.float32),
        scratch_types=[
            pltpu.VMEM((C,), jnp.int32),
            pltpu.VMEM((C,), jnp.int32),
            pltpu.VMEM((C, H_), jnp.float32),
            pltpu.VMEM((C, H_), jnp.float32),
            pltpu.VMEM((H_,), jnp.float32),
            pltpu.VMEM((16,), jnp.float32),
            pltpu.VMEM((C,), jnp.float32),
            pltpu.SemaphoreType.DMA,
            pltpu.SemaphoreType.DMA,
        ],
        compiler_params=_sc_params(),
    )
    def head(p_hbm, q_hbm, row_hbm, col_hbm, w2_hbm, b2_hbm, out_hbm,
             ridx, cidx, pbuf, qbuf, w2v, b2v_v, obuf, sem1, sem2):
        wid = lax.axis_index("s") * _NC + lax.axis_index("c")
        base = wid * per_w
        pltpu.sync_copy(w2_hbm, w2v)
        pltpu.sync_copy(b2_hbm, b2v_v)

        @pl.loop(0, n_chunks)
        def _chunk(ci):
            off = base + ci * C
            pltpu.sync_copy(row_hbm.at[pl.ds(off, C)], ridx)
            pltpu.sync_copy(col_hbm.at[pl.ds(off, C)], cidx)
            cp1 = pltpu.async_copy(p_hbm.at[ridx], pbuf, sem1)
            cp2 = pltpu.async_copy(q_hbm.at[cidx], qbuf, sem2)
            cp1.wait()
            cp2.wait()

            @pl.loop(0, C, step=16)
            def _grp(j):
                eids = lax.iota(jnp.int32, 16) + j

                @pl.loop(0, H_, init_carry=b2v_v[...], unroll=8)
                def _feat(kk, acc):
                    kkv = jnp.full((16,), kk, jnp.int32)
                    pk = plsc.load_gather(pbuf, [eids, kkv])
                    qk = plsc.load_gather(qbuf, [eids, kkv])
                    wk = plsc.load_gather(w2v, [kkv])
                    hv = jnp.maximum(pk + qk, 0.0)
                    return acc + hv * wk

                s = _feat
                obuf[pl.ds(j, 16)] = 1.0 / (1.0 + jnp.exp(-s))

            pltpu.sync_copy(obuf, out_hbm.at[pl.ds(off, C)])

    return head(P, Q, row, col, w2, b2v)


def _gcn_dense(agg, Xc, pW, sW, b, relu=True):
    N_, Din = Xc.shape
    Dout = pW.shape[1]
    BN = 1000

    def body(agg_ref, x_ref, pw_ref, sw_ref, b_ref, o_ref):
        acc = jnp.dot(agg_ref[...], pw_ref[...], preferred_element_type=jnp.float32)
        acc = acc + jnp.dot(x_ref[...], sw_ref[...], preferred_element_type=jnp.float32)
        acc = acc + b_ref[...]
        if relu:
            acc = jnp.maximum(acc, 0.0)
        o_ref[...] = acc

    return pl.pallas_call(
        body,
        grid=(N_ // BN,),
        in_specs=[
            pl.BlockSpec((BN, Din), lambda i: (i, 0)),
            pl.BlockSpec((BN, Din), lambda i: (i, 0)),
            pl.BlockSpec((Din, Dout), lambda i: (0, 0)),
            pl.BlockSpec((Din, Dout), lambda i: (0, 0)),
            pl.BlockSpec((1, Dout), lambda i: (0, 0)),
        ],
        out_specs=pl.BlockSpec((BN, Dout), lambda i: (i, 0)),
        out_shape=jax.ShapeDtypeStruct((N_, Dout), jnp.float32),
    )(agg, Xc, pW, sW, b.reshape(1, -1))


def _edge_head(PR, QC, w2row, b2):
    E_ = PR.shape[0]
    H_ = PR.shape[1]
    BE = 512

    def body(pr_ref, qc_ref, w2_ref, b2_ref, o_ref):
        h = jnp.maximum(pr_ref[...] + qc_ref[...], 0.0)
        s = jnp.sum(h * w2_ref[...], axis=1) + b2_ref[0, 0]
        o_ref[...] = jax.nn.sigmoid(s)

    return pl.pallas_call(
        body,
        grid=(E_ // BE,),
        in_specs=[
            pl.BlockSpec((BE, H_), lambda i: (i, 0)),
            pl.BlockSpec((BE, H_), lambda i: (i, 0)),
            pl.BlockSpec((1, H_), lambda i: (0, 0)),
            pl.BlockSpec((1, 1), lambda i: (0, 0)),
        ],
        out_specs=pl.BlockSpec((BE,), lambda i: (i,)),
        out_shape=jax.ShapeDtypeStruct((E_,), jnp.float32),
    )(PR, QC, w2row, b2.reshape(1, 1))


def kernel(X, edge_index, edge_weight,
           pass_W1, pass_b1, self_W1, self_b1,
           pass_W2, pass_b2, self_W2, self_b2,
           lin_W1, lin_b1, lin_W2, lin_b2):
    N_ = X.shape[0]
    H_ = pass_W1.shape[1]
    row = edge_index[0]
    col = edge_index[1]

    msgs1 = edge_weight[:, None] * jnp.take(X, col, axis=0)
    agg1 = jax.ops.segment_sum(msgs1, row, num_segments=N_)
    X1 = _gcn_dense(agg1, X, pass_W1, self_W1, pass_b1 + self_b1)

    msgs2 = edge_weight[:, None] * jnp.take(X1, col, axis=0)
    agg2 = jax.ops.segment_sum(msgs2, row, num_segments=N_)
    X2 = _gcn_dense(agg2, X1, pass_W2, self_W2, pass_b2 + self_b2)

    W_pq = jnp.concatenate([lin_W1[:H_], lin_W1[H_:]], axis=1)
    b_pq = jnp.concatenate([lin_b1, jnp.zeros_like(lin_b1)])

    BN = 1000

    def pq_body(x_ref, w_ref, b_ref, o_ref):
        o_ref[...] = jnp.dot(x_ref[...], w_ref[...], preferred_element_type=jnp.float32) + b_ref[...]

    PQmat = pl.pallas_call(
        pq_body,
        grid=(N_ // BN,),
        in_specs=[
            pl.BlockSpec((BN, H_), lambda i: (i, 0)),
            pl.BlockSpec((H_, 2 * H_), lambda i: (0, 0)),
            pl.BlockSpec((1, 2 * H_), lambda i: (0, 0)),
        ],
        out_specs=pl.BlockSpec((BN, 2 * H_), lambda i: (i, 0)),
        out_shape=jax.ShapeDtypeStruct((N_, 2 * H_), jnp.float32),
    )(X2, W_pq, b_pq.reshape(1, -1))
    P = PQmat[:, :H_]
    Q = PQmat[:, H_:]

    b2v = jnp.full((16,), lin_b2[0], jnp.float32)
    return _edge_head_sc(P, Q, row, col, lin_W2[:, 0], b2v)


# speedup vs baseline: 1.0569x; 1.0569x over previous
import dataclasses
import functools

import jax
import jax.numpy as jnp
from jax import lax
from jax.experimental import pallas as pl
from jax.experimental.pallas import tpu as pltpu
from jax.experimental.pallas import tpu_sc as plsc

_NC = 2
_NS = 16
_NW = _NC * _NS


def _sc_params():
    cp = pltpu.CompilerParams()
    if "needs_layout_passes" in pltpu.CompilerParams.__dataclass_fields__:
        cp = dataclasses.replace(cp, needs_layout_passes=False)
    return cp


def _edge_head_sc(P, Q, row, col, w2, b2v):
    E_ = row.shape[0]
    H_ = P.shape[1]
    C = 80
    G = C // 16
    per_w = E_ // _NW
    n_chunks = per_w // C
    mesh = plsc.VectorSubcoreMesh(core_axis_name="c", subcore_axis_name="s")

    @functools.partial(
        pl.kernel, mesh=mesh,
        out_type=jax.ShapeDtypeStruct((E_,), jnp.float32),
        scratch_types=[
            pltpu.VMEM((per_w,), jnp.int32),
            pltpu.VMEM((per_w,), jnp.int32),
            pltpu.VMEM((C, H_), jnp.float32),
            pltpu.VMEM((C, H_), jnp.float32),
            pltpu.VMEM((C, H_), jnp.float32),
            pltpu.VMEM((C, H_), jnp.float32),
            pltpu.VMEM((H_,), jnp.float32),
            pltpu.VMEM((16,), jnp.float32),
            pltpu.VMEM((per_w,), jnp.float32),
            pltpu.SemaphoreType.DMA,
            pltpu.SemaphoreType.DMA,
            pltpu.SemaphoreType.DMA,
            pltpu.SemaphoreType.DMA,
        ],
        compiler_params=_sc_params(),
    )
    def head(p_hbm, q_hbm, row_hbm, col_hbm, w2_hbm, b2_hbm, out_hbm,
             ridxs, cidxs, pb0, qb0, pb1, qb1, w2v, b2v_v, obuf,
             semp0, semq0, semp1, semq1):
        wid = lax.axis_index("s") * _NC + lax.axis_index("c")
        base = wid * per_w
        pltpu.sync_copy(row_hbm.at[pl.ds(base, per_w)], ridxs)
        pltpu.sync_copy(col_hbm.at[pl.ds(base, per_w)], cidxs)
        pltpu.sync_copy(w2_hbm, w2v)
        pltpu.sync_copy(b2_hbm, b2v_v)

        bufs = ((pb0, qb0, semp0, semq0), (pb1, qb1, semp1, semq1))

        def start(ci, b):
            pb, qb, sp, sq = bufs[b]
            pltpu.make_async_copy(
                p_hbm.at[ridxs.at[pl.ds(ci * C, C)]], pb, sp).start()
            pltpu.make_async_copy(
                q_hbm.at[cidxs.at[pl.ds(ci * C, C)]], qb, sq).start()

        def wait(ci, b):
            pb, qb, sp, sq = bufs[b]
            pltpu.make_async_copy(
                p_hbm.at[ridxs.at[pl.ds(ci * C, C)]], pb, sp).wait()
            pltpu.make_async_copy(
                q_hbm.at[cidxs.at[pl.ds(ci * C, C)]], qb, sq).wait()

        def compute(ci, b):
            pb, qb, _, _ = bufs[b]
            eids = tuple(lax.iota(jnp.int32, 16) + (16 * g) for g in range(G))
            init = tuple(b2v_v[...] for _ in range(G))

            @pl.loop(0, H_, init_carry=init, unroll=4)
            def _feat(kk, accs):
                kkv = jnp.full((16,), kk, jnp.int32)
                wk = plsc.load_gather(w2v, [kkv])
                out = []
                for g in range(G):
                    pk = plsc.load_gather(pb, [eids[g], kkv])
                    qk = plsc.load_gather(qb, [eids[g], kkv])
                    hv = jnp.maximum(pk + qk, 0.0)
                    out.append(accs[g] + hv * wk)
                return tuple(out)

            accs = _feat
            for g in range(G):
                s = accs[g]
                obuf[pl.ds(ci * C + 16 * g, 16)] = 1.0 / (1.0 + jnp.exp(-s))

        start(0, 0)

        @pl.loop(0, n_chunks - 1, step=2)
        def _pair(ci):
            start(ci + 1, 1)
            wait(ci, 0)
            compute(ci, 0)

            @pl.when(ci + 2 < n_chunks)
            def _():
                start(ci + 2, 0)

            wait(ci + 1, 1)
            compute(ci + 1, 1)

        wait(n_chunks - 1, 0)
        compute(n_chunks - 1, 0)
        pltpu.sync_copy(obuf, out_hbm.at[pl.ds(base, per_w)])

    return head(P, Q, row, col, w2, b2v)


def _gcn_dense(agg, Xc, pW, sW, b, relu=True):
    N_, Din = Xc.shape
    Dout = pW.shape[1]
    BN = 1000

    def body(agg_ref, x_ref, pw_ref, sw_ref, b_ref, o_ref):
        acc = jnp.dot(agg_ref[...], pw_ref[...], preferred_element_type=jnp.float32)
        acc = acc + jnp.dot(x_ref[...], sw_ref[...], preferred_element_type=jnp.float32)
        acc = acc + b_ref[...]
        if relu:
            acc = jnp.maximum(acc, 0.0)
        o_ref[...] = acc

    return pl.pallas_call(
        body,
        grid=(N_ // BN,),
        in_specs=[
            pl.BlockSpec((BN, Din), lambda i: (i, 0)),
            pl.BlockSpec((BN, Din), lambda i: (i, 0)),
            pl.BlockSpec((Din, Dout), lambda i: (0, 0)),
            pl.BlockSpec((Din, Dout), lambda i: (0, 0)),
            pl.BlockSpec((1, Dout), lambda i: (0, 0)),
        ],
        out_specs=pl.BlockSpec((BN, Dout), lambda i: (i, 0)),
        out_shape=jax.ShapeDtypeStruct((N_, Dout), jnp.float32),
    )(agg, Xc, pW, sW, b.reshape(1, -1))


def _edge_head(PR, QC, w2row, b2):
    E_ = PR.shape[0]
    H_ = PR.shape[1]
    BE = 512

    def body(pr_ref, qc_ref, w2_ref, b2_ref, o_ref):
        h = jnp.maximum(pr_ref[...] + qc_ref[...], 0.0)
        s = jnp.sum(h * w2_ref[...], axis=1) + b2_ref[0, 0]
        o_ref[...] = jax.nn.sigmoid(s)

    return pl.pallas_call(
        body,
        grid=(E_ // BE,),
        in_specs=[
            pl.BlockSpec((BE, H_), lambda i: (i, 0)),
            pl.BlockSpec((BE, H_), lambda i: (i, 0)),
            pl.BlockSpec((1, H_), lambda i: (0, 0)),
            pl.BlockSpec((1, 1), lambda i: (0, 0)),
        ],
        out_specs=pl.BlockSpec((BE,), lambda i: (i,)),
        out_shape=jax.ShapeDtypeStruct((E_,), jnp.float32),
    )(PR, QC, w2row, b2.reshape(1, 1))


def kernel(X, edge_index, edge_weight,
           pass_W1, pass_b1, self_W1, self_b1,
           pass_W2, pass_b2, self_W2, self_b2,
           lin_W1, lin_b1, lin_W2, lin_b2):
    N_ = X.shape[0]
    H_ = pass_W1.shape[1]
    row = edge_index[0]
    col = edge_index[1]

    msgs1 = edge_weight[:, None] * jnp.take(X, col, axis=0)
    agg1 = jax.ops.segment_sum(msgs1, row, num_segments=N_)
    X1 = _gcn_dense(agg1, X, pass_W1, self_W1, pass_b1 + self_b1)

    msgs2 = edge_weight[:, None] * jnp.take(X1, col, axis=0)
    agg2 = jax.ops.segment_sum(msgs2, row, num_segments=N_)
    X2 = _gcn_dense(agg2, X1, pass_W2, self_W2, pass_b2 + self_b2)

    W_pq = jnp.concatenate([lin_W1[:H_], lin_W1[H_:]], axis=1)
    b_pq = jnp.concatenate([lin_b1, jnp.zeros_like(lin_b1)])

    BN = 1000

    def pq_body(x_ref, w_ref, b_ref, o_ref):
        o_ref[...] = jnp.dot(x_ref[...], w_ref[...], preferred_element_type=jnp.float32) + b_ref[...]

    PQmat = pl.pallas_call(
        pq_body,
        grid=(N_ // BN,),
        in_specs=[
            pl.BlockSpec((BN, H_), lambda i: (i, 0)),
            pl.BlockSpec((H_, 2 * H_), lambda i: (0, 0)),
            pl.BlockSpec((1, 2 * H_), lambda i: (0, 0)),
        ],
        out_specs=pl.BlockSpec((BN, 2 * H_), lambda i: (i, 0)),
        out_shape=jax.ShapeDtypeStruct((N_, 2 * H_), jnp.float32),
    )(X2, W_pq, b_pq.reshape(1, -1))
    P = PQmat[:, :H_]
    Q = PQmat[:, H_:]

    b2v = jnp.full((16,), lin_b2[0], jnp.float32)
    return _edge_head_sc(P, Q, row, col, lin_W2[:, 0], b2v)


# speedup vs baseline: 1.4817x; 1.4019x over previous
import dataclasses
import functools

import jax
import jax.numpy as jnp
from jax import lax
from jax.experimental import pallas as pl
from jax.experimental.pallas import tpu as pltpu
from jax.experimental.pallas import tpu_sc as plsc

_NC = 2
_NS = 16
_NW = _NC * _NS


def _sc_params():
    cp = pltpu.CompilerParams()
    if "needs_layout_passes" in pltpu.CompilerParams.__dataclass_fields__:
        cp = dataclasses.replace(cp, needs_layout_passes=False)
    return cp


def _dense_multi(pairs, addend, bias, relu, out_widths):
    N_ = pairs[0][0].shape[0]
    Dout = pairs[0][1].shape[1]
    BN = 1000
    n_in = len(pairs)
    has_add = addend is not None
    has_bias = bias is not None
    n_out = len(out_widths)

    def body(*refs):
        in_refs = refs[:-n_out]
        out_refs = refs[-n_out:]
        acc = None
        for i in range(n_in):
            d = jnp.dot(in_refs[2 * i][...], in_refs[2 * i + 1][...],
                        preferred_element_type=jnp.float32)
            acc = d if acc is None else acc + d
        idx = 2 * n_in
        if has_add:
            acc = acc + in_refs[idx][...]
            idx += 1
        if has_bias:
            acc = acc + in_refs[idx][...]
        if relu:
            acc = jnp.maximum(acc, 0.0)
        off = 0
        for o_ref, w in zip(out_refs, out_widths):
            o_ref[...] = acc[:, off:off + w]
            off += w

    in_specs = []
    args = []
    for (x, w) in pairs:
        kin = x.shape[1]
        in_specs.append(pl.BlockSpec((BN, kin), lambda i: (i, 0)))
        args.append(x)
        in_specs.append(pl.BlockSpec(w.shape, lambda i: (0, 0)))
        args.append(w)
    if has_add:
        in_specs.append(pl.BlockSpec((BN, Dout), lambda i: (i, 0)))
        args.append(addend)
    if has_bias:
        in_specs.append(pl.BlockSpec((1, Dout), lambda i: (0, 0)))
        args.append(bias.reshape(1, -1))
    out_specs = [pl.BlockSpec((BN, w), lambda i: (i, 0)) for w in out_widths]
    out_shape = [jax.ShapeDtypeStruct((N_, w), jnp.float32) for w in out_widths]
    res = pl.pallas_call(
        body,
        grid=(N_ // BN,),
        in_specs=in_specs,
        out_specs=out_specs,
        out_shape=out_shape,
    )(*args)
    return res if n_out > 1 else res[0]


_SEG_C = 40
_SEG_NBUF = 5


def _seg_sum_sc(table0, table1, row, col, w, zeros, edge_split):
    N_, D_ = table0.shape
    C = _SEG_C
    E_ = row.shape[0]
    per_t = E_ // (_NW if edge_split else _NS)
    nch = per_t // C
    assert nch % _SEG_NBUF == 0 and C % 8 == 0
    mesh = plsc.VectorSubcoreMesh(core_axis_name="c", subcore_axis_name="s")

    out_sds = jax.ShapeDtypeStruct((N_, D_), jnp.float32)
    NB = _SEG_NBUF

    @functools.partial(
        pl.kernel, mesh=mesh,
        out_type=(out_sds, out_sds),
        scratch_types=[
            pltpu.VMEM_SHARED((N_, D_), jnp.float32),
        ] + [pltpu.VMEM((C, D_), jnp.float32) for _ in range(NB)]
          + [pltpu.VMEM((C,), jnp.int32) for _ in range(NB)]
          + [pltpu.VMEM((C,), jnp.int32) for _ in range(NB)]
          + [pltpu.VMEM((C,), jnp.float32) for _ in range(NB)]
          + [pltpu.SemaphoreType.DMA for _ in range(3 * NB)],
        compiler_params=_sc_params(),
    )
    def seg(t0_hbm, t1_hbm, row_hbm, col_hbm, w_hbm, z_hbm, o0_hbm, o1_hbm,
            acc, *bufs_sems):
        gbuf = bufs_sems[:NB]
        idxb = bufs_sems[NB:2 * NB]
        cidb = bufs_sems[2 * NB:3 * NB]
        wvb = bufs_sems[3 * NB:4 * NB]
        semg = bufs_sems[4 * NB:5 * NB]
        semi = bufs_sems[5 * NB:6 * NB]
        sems = bufs_sems[6 * NB:]
        cid = lax.axis_index("c")
        sid = lax.axis_index("s")
        wid = sid * _NC + cid
        base = (wid if edge_split else sid) * per_t

        @pl.when(sid == 0)
        def _zero():
            pltpu.sync_copy(z_hbm, acc)

        plsc.subcore_barrier()

        def pipe(table_ref, out_ref):
            def i_start(cc, b):
                off = base + cc * C
                pltpu.async_copy(row_hbm.at[pl.ds(off, C)], idxb[b], semi[b])
                pltpu.async_copy(col_hbm.at[pl.ds(off, C)], cidb[b], semi[b])
                pltpu.async_copy(w_hbm.at[pl.ds(off, C)], wvb[b], semi[b])

            def i_wait(cc, b):
                off = base + cc * C
                pltpu.make_async_copy(
                    row_hbm.at[pl.ds(off, C)], idxb[b], semi[b]).wait()
                pltpu.make_async_copy(
                    col_hbm.at[pl.ds(off, C)], cidb[b], semi[b]).wait()
                pltpu.make_async_copy(
                    w_hbm.at[pl.ds(off, C)], wvb[b], semi[b]).wait()

            def g_start(cc, b):
                pltpu.async_copy(table_ref.at[cidb[b]], gbuf[b], semg[b])

            def g_wait(cc, b):
                pltpu.make_async_copy(
                    table_ref.at[cidb[b]], gbuf[b], semg[b]).wait()

            def s_start(cc, b):
                pltpu.async_copy(gbuf[b], acc.at[idxb[b]], sems[b], add=True)

            def s_wait(cc, b):
                pltpu.make_async_copy(
                    gbuf[b], acc.at[idxb[b]], sems[b]).wait()

            i_start(0, 0)
            i_start(1, 1)
            i_wait(0, 0)
            g_start(0, 0)

            @pl.loop(0, nch, step=NB)
            def _ring(ci):
                for t in range(NB):
                    cc = ci + t
                    b = t
                    b1 = (t + 1) % NB
                    b2 = (t + 2) % NB

                    @pl.when(cc >= 3)
                    def _():
                        s_wait(cc - 3, b2)

                    @pl.when(cc + 2 < nch)
                    def _():
                        i_start(cc + 2, b2)

                    @pl.when(cc + 1 < nch)
                    def _():
                        i_wait(cc + 1, b1)
                        g_start(cc + 1, b1)

                    g_wait(cc, b)
                    buf = gbuf[b]

                    @pl.loop(0, C, unroll=2)
                    def _scale(e):
                        ws = plsc.load_gather(
                            wvb[b], [jnp.full((16,), e, jnp.int32)])
                        for g in range(D_ // 16):
                            sl = pl.ds(16 * g, 16)
                            buf[e, sl] = buf[e, sl] * ws

                    s_start(cc, b)

            s_wait(nch - 3, (nch - 3) % NB)
            s_wait(nch - 2, (nch - 2) % NB)
            s_wait(nch - 1, (nch - 1) % NB)
            plsc.subcore_barrier()

            @pl.when(sid == 0)
            def _writeback():
                pltpu.sync_copy(acc, out_ref)

        @pl.when(cid == 0)
        def _c0():
            pipe(t0_hbm, o0_hbm)

        @pl.when(cid == 1)
        def _c1():
            pipe(t1_hbm, o1_hbm)

    return seg(table0, table1, row, col, w, zeros)


def _edge_head_sc(P, Q, row, col, w2, b2v):
    E_ = row.shape[0]
    H_ = P.shape[1]
    C = 80
    G = C // 16
    per_w = E_ // _NW
    n_chunks = per_w // C
    mesh = plsc.VectorSubcoreMesh(core_axis_name="c", subcore_axis_name="s")

    @functools.partial(
        pl.kernel, mesh=mesh,
        out_type=jax.ShapeDtypeStruct((E_,), jnp.float32),
        scratch_types=[
            pltpu.VMEM((per_w,), jnp.int32),
            pltpu.VMEM((per_w,), jnp.int32),
            pltpu.VMEM((C, H_), jnp.float32),
            pltpu.VMEM((C, H_), jnp.float32),
            pltpu.VMEM((C, H_), jnp.float32),
            pltpu.VMEM((C, H_), jnp.float32),
            pltpu.VMEM((H_,), jnp.float32),
            pltpu.VMEM((16,), jnp.float32),
            pltpu.VMEM((per_w,), jnp.float32),
            pltpu.SemaphoreType.DMA,
            pltpu.SemaphoreType.DMA,
            pltpu.SemaphoreType.DMA,
            pltpu.SemaphoreType.DMA,
        ],
        compiler_params=_sc_params(),
    )
    def head(p_hbm, q_hbm, row_hbm, col_hbm, w2_hbm, b2_hbm, out_hbm,
             ridxs, cidxs, pb0, qb0, pb1, qb1, w2v, b2v_v, obuf,
             semp0, semq0, semp1, semq1):
        wid = lax.axis_index("s") * _NC + lax.axis_index("c")
        base = wid * per_w
        pltpu.sync_copy(row_hbm.at[pl.ds(base, per_w)], ridxs)
        pltpu.sync_copy(col_hbm.at[pl.ds(base, per_w)], cidxs)
        pltpu.sync_copy(w2_hbm, w2v)
        pltpu.sync_copy(b2_hbm, b2v_v)

        bufs = ((pb0, qb0, semp0, semq0), (pb1, qb1, semp1, semq1))

        def start(ci, b):
            pb, qb, sp, sq = bufs[b]
            pltpu.make_async_copy(
                p_hbm.at[ridxs.at[pl.ds(ci * C, C)]], pb, sp).start()
            pltpu.make_async_copy(
                q_hbm.at[cidxs.at[pl.ds(ci * C, C)]], qb, sq).start()

        def wait(ci, b):
            pb, qb, sp, sq = bufs[b]
            pltpu.make_async_copy(
                p_hbm.at[ridxs.at[pl.ds(ci * C, C)]], pb, sp).wait()
            pltpu.make_async_copy(
                q_hbm.at[cidxs.at[pl.ds(ci * C, C)]], qb, sq).wait()

        def compute(ci, b):
            pb, qb, _, _ = bufs[b]
            eids = tuple(lax.iota(jnp.int32, 16) + (16 * g) for g in range(G))
            init = tuple(b2v_v[...] for _ in range(G))

            @pl.loop(0, H_, init_carry=init, unroll=4)
            def _feat(kk, accs):
                kkv = jnp.full((16,), kk, jnp.int32)
                wk = plsc.load_gather(w2v, [kkv])
                out = []
                for g in range(G):
                    pk = plsc.load_gather(pb, [eids[g], kkv])
                    qk = plsc.load_gather(qb, [eids[g], kkv])
                    hv = jnp.maximum(pk + qk, 0.0)
                    out.append(accs[g] + hv * wk)
                return tuple(out)

            accs = _feat
            for g in range(G):
                s = accs[g]
                obuf[pl.ds(ci * C + 16 * g, 16)] = 1.0 / (1.0 + jnp.exp(-s))

        start(0, 0)

        @pl.loop(0, n_chunks - 1, step=2)
        def _pair(ci):
            start(ci + 1, 1)
            wait(ci, 0)
            compute(ci, 0)

            @pl.when(ci + 2 < n_chunks)
            def _():
                start(ci + 2, 0)

            wait(ci + 1, 1)
            compute(ci + 1, 1)

        wait(n_chunks - 1, 0)
        compute(n_chunks - 1, 0)
        pltpu.sync_copy(obuf, out_hbm.at[pl.ds(base, per_w)])

    return head(P, Q, row, col, w2, b2v)


def _edge_head_tc(PR, QC, w2row, b2):
    E_, H_ = PR.shape
    BE = 512

    def body(pr_ref, qc_ref, w2_ref, b2_ref, o_ref):
        h = jnp.maximum(pr_ref[...] + qc_ref[...], 0.0)
        s = jnp.sum(h * w2_ref[...], axis=1) + b2_ref[0, 0]
        o_ref[...] = jax.nn.sigmoid(s)

    return pl.pallas_call(
        body,
        grid=(E_ // BE,),
        in_specs=[
            pl.BlockSpec((BE, H_), lambda i: (i, 0)),
            pl.BlockSpec((BE, H_), lambda i: (i, 0)),
            pl.BlockSpec((1, H_), lambda i: (0, 0)),
            pl.BlockSpec((1, 1), lambda i: (0, 0)),
        ],
        out_specs=pl.BlockSpec((BE,), lambda i: (i,)),
        out_shape=jax.ShapeDtypeStruct((E_,), jnp.float32),
    )(PR, QC, w2row, b2.reshape(1, 1))


def kernel(X, edge_index, edge_weight,
           pass_W1, pass_b1, self_W1, self_b1,
           pass_W2, pass_b2, self_W2, self_b2,
           lin_W1, lin_b1, lin_W2, lin_b2):
    N_ = X.shape[0]
    H_ = pass_W1.shape[1]
    row = edge_index[0]
    col = edge_index[1]
    zeros = jnp.zeros((N_, 128), jnp.float32)

    msgs1 = edge_weight[:, None] * jnp.take(X, col, axis=0)
    agg1 = jax.ops.segment_sum(msgs1, row, num_segments=N_)
    X1a, X1b = _dense_multi([(agg1, pass_W1), (X, self_W1)], None,
                            pass_b1 + self_b1, True, [128, 128])

    h0, h1 = _seg_sum_sc(X1a, X1b, row, col, edge_weight, zeros,
                         edge_split=False)
    pre2 = _dense_multi([(X1a, self_W2[:128]), (X1b, self_W2[128:])], None,
                        pass_b2 + self_b2, False, [H_])
    X2a, X2b = _dense_multi([(h0, pass_W2[:128]), (h1, pass_W2[128:])], pre2,
                            None, True, [128, 128])

    W_pq = jnp.concatenate([lin_W1[:H_], lin_W1[H_:]], axis=1)
    b_pq = jnp.concatenate([lin_b1, jnp.zeros_like(lin_b1)])
    P, Q = _dense_multi([(X2a, W_pq[:128]), (X2b, W_pq[128:])], None, b_pq,
                        False, [H_, H_])

    PR = jnp.take(P, row, axis=0)
    QC = jnp.take(Q, col, axis=0)
    return _edge_head_tc(PR, QC, lin_W2[:, 0].reshape(1, -1), lin_b2[0])
